# baseline (speedup 1.0000x reference)
import functools

import jax
import jax.numpy as jnp
from jax import lax
from jax.experimental import pallas as pl
from jax.experimental.pallas import tpu as pltpu
from jax.experimental.pallas import tpu_sc as plsc

B, L, H, A, E = 4, 8192, 768, 64, 8
NUM_T_BUCKETS, NUM_TOKEN_TYPES = 8, 4

N_TOK = B * L
N_SC = 2048
N_TC = N_TOK - N_SC
TL = 2048

NW = 32
PER_TEC = N_SC // NW
GROUP = 16
T_SUB = 2
N_GROUPS = PER_TEC // GROUP
CHUNKS = H // 16


def _tc_router_block(h_ref, tb_ref, ty_ref, tw_ref, tyw_ref, rw_ref, rb_ref,
                     out_ref):
    logits = jax.lax.dot_general(
        h_ref[...], rw_ref[:, :H], (((1,), (1,)), ((), ())),
        preferred_element_type=jnp.float32)

    lut_t = jax.lax.dot_general(
        tw_ref[...], rw_ref[:, H:H + A], (((1,), (1,)), ((), ())),
        preferred_element_type=jnp.float32)
    lut_ty = jax.lax.dot_general(
        tyw_ref[...], rw_ref[:, H + A:], (((1,), (1,)), ((), ())),
        preferred_element_type=jnp.float32)

    tb = tb_ref[0, 0, :]
    ty = ty_ref[0, 0, :]
    oh_t = (tb[:, None] == jax.lax.broadcasted_iota(
        jnp.int32, (TL, NUM_T_BUCKETS), 1)).astype(jnp.float32)
    oh_ty = (ty[:, None] == jax.lax.broadcasted_iota(
        jnp.int32, (TL, NUM_TOKEN_TYPES), 1)).astype(jnp.float32)
    logits = logits + jax.lax.dot_general(
        oh_t, lut_t, (((1,), (0,)), ((), ())),
        preferred_element_type=jnp.float32)
    logits = logits + jax.lax.dot_general(
        oh_ty, lut_ty, (((1,), (0,)), ((), ())),
        preferred_element_type=jnp.float32)
    logits = logits + rb_ref[0, :][None, :]

    m = jnp.max(logits, axis=1, keepdims=True)
    e = jnp.exp(logits - m)
    out_ref[...] = e / jnp.sum(e, axis=1, keepdims=True)


def _tc_router(h2, tb3, ty3, tw, tyw, rw, rb2):
    n_blk = N_TC // TL
    return pl.pallas_call(
        _tc_router_block,
        grid=(n_blk,),
        in_specs=[
            pl.BlockSpec((TL, H), lambda i: (i, 0)),
            pl.BlockSpec((1, 1, TL), lambda i: (i, 0, 0)),
            pl.BlockSpec((1, 1, TL), lambda i: (i, 0, 0)),
            pl.BlockSpec((NUM_T_BUCKETS, A), lambda i: (0, 0)),
            pl.BlockSpec((NUM_TOKEN_TYPES, A), lambda i: (0, 0)),
            pl.BlockSpec((E, H + 2 * A), lambda i: (0, 0)),
            pl.BlockSpec((1, E), lambda i: (0, 0)),
        ],
        out_specs=pl.BlockSpec((TL, E), lambda i: (i, 0)),
        out_shape=jax.ShapeDtypeStruct((N_TC, E), jnp.float32),
    )(h2, tb3, ty3, tw, tyw, rw, rb2)


def _lut_kernel(tw_ref, tyw_ref, rw_ref, rb_ref, lt_ref, lty_ref):
    lt = jax.lax.dot_general(
        tw_ref[...], rw_ref[:, H:H + A], (((1,), (1,)), ((), ())),
        preferred_element_type=jnp.float32)
    lt_ref[...] = lt + rb_ref[0, :][None, :]
    lty_ref[...] = jax.lax.dot_general(
        tyw_ref[...], rw_ref[:, H + A:], (((1,), (1,)), ((), ())),
        preferred_element_type=jnp.float32)


def _make_luts(tw, tyw, rw, rb2):
    return pl.pallas_call(
        _lut_kernel,
        out_shape=(jax.ShapeDtypeStruct((NUM_T_BUCKETS, E), jnp.float32),
                   jax.ShapeDtypeStruct((NUM_TOKEN_TYPES, E), jnp.float32)),
    )(tw, tyw, rw, rb2)


def _sc_router(h_hbm, tb_hbm, ty_hbm, wh_hbm, lutt_hbm, lutty_hbm, out_hbm,
               hbuf, wbuf, ltbuf, ltybuf, tbuf, tybuf, accbuf, obuf, sem):
    wid = lax.axis_index("c") * 16 + lax.axis_index("s")
    tok0 = N_TC + wid * PER_TEC

    cp = pltpu.make_async_copy(h_hbm.at[pl.ds(tok0, PER_TEC), :], hbuf, sem)
    cp.start()
    pltpu.sync_copy(wh_hbm, wbuf)
    pltpu.sync_copy(lutt_hbm, ltbuf)
    pltpu.sync_copy(lutty_hbm, ltybuf)
    pltpu.sync_copy(tb_hbm.at[pl.ds(tok0, PER_TEC)], tbuf)
    pltpu.sync_copy(ty_hbm.at[pl.ds(tok0, PER_TEC)], tybuf)
    cp.wait()

    iv = lax.iota(jnp.int32, 16)
    iv16 = iv * 16
    iv8 = iv * 8

    def group_body(g, carry):
        def sub_body(sb, carry2):
            tok = g * GROUP + sb * T_SUB
            acc = [jnp.zeros((16,), jnp.float32) for _ in range(E * T_SUB)]
            for c in range(CHUNKS):
                base = c * 16
                hv = [hbuf[tok + t, base:base + 16] for t in range(T_SUB)]
                for e in range(E):
                    wv = wbuf[e, base:base + 16]
                    for t in range(T_SUB):
                        i = e * T_SUB + t
                        acc[i] = acc[i] + hv[t] * wv
            for e in range(E):
                for t in range(T_SUB):
                    off = e * (GROUP * 16) + sb * (T_SUB * 16) + t * 16
                    accbuf[pl.ds(off, 16)] = acc[e * T_SUB + t]
            return carry2

        lax.fori_loop(0, GROUP // T_SUB, sub_body, 0)

        tbv = tbuf[pl.ds(g * GROUP, 16)]
        tyv = tybuf[pl.ds(g * GROUP, 16)]
        logits = []
        for e in range(E):
            le = plsc.load_gather(accbuf, [iv16 + e * (GROUP * 16)])
            for p in range(1, 16):
                le = le + plsc.load_gather(
                    accbuf, [iv16 + (e * (GROUP * 16) + p)])
            le = le + plsc.load_gather(ltbuf, [tbv * E + e])
            le = le + plsc.load_gather(ltybuf, [tyv * E + e])
            logits.append(le)

        m = logits[0]
        for e in range(1, E):
            m = jnp.maximum(m, logits[e])
        es = [jnp.exp(le - m) for le in logits]
        s = es[0]
        for e in range(1, E):
            s = s + es[e]
        r = 1.0 / s
        obase = g * (GROUP * E)
        for e in range(E):
            plsc.store_scatter(obuf, [iv8 + (obase + e)], es[e] * r)
        return carry

    lax.fori_loop(0, N_GROUPS, group_body, 0)
    pltpu.sync_copy(obuf, out_hbm.at[pl.ds(wid * PER_TEC * E, PER_TEC * E)])


def _sc_router_call(h2, tbf, tyf, wh, lutt_f, lutty_f):
    mesh = plsc.VectorSubcoreMesh(core_axis_name="c", subcore_axis_name="s",
                                  num_cores=2, num_subcores=16)
    fn = pl.kernel(
        _sc_router,
        out_type=jax.ShapeDtypeStruct((N_SC * E,), jnp.float32),
        mesh=mesh,
        compiler_params=pltpu.CompilerParams(needs_layout_passes=False),
        cost_estimate=pl.CostEstimate(
            flops=2 * N_SC * H * E, bytes_accessed=N_SC * H * 4,
            transcendentals=N_SC * E),
        scratch_types=[
            pltpu.VMEM((PER_TEC, H), jnp.float32),
            pltpu.VMEM((E, H), jnp.float32),
            pltpu.VMEM((NUM_T_BUCKETS * E,), jnp.float32),
            pltpu.VMEM((NUM_TOKEN_TYPES * E,), jnp.float32),
            pltpu.VMEM((PER_TEC,), jnp.int32),
            pltpu.VMEM((PER_TEC,), jnp.int32),
            pltpu.VMEM((E * GROUP * 16,), jnp.float32),
            pltpu.VMEM((PER_TEC * E,), jnp.float32),
            pltpu.SemaphoreType.DMA,
        ],
    )
    return fn(h2, tbf, tyf, wh, lutt_f, lutty_f)


@functools.partial(jax.jit, static_argnames=())
def kernel(hidden_states, t_bucket, token_type, t_embed_weight,
           type_embed_weight, router_w, router_b):
    h2 = hidden_states.reshape(N_TOK, H)
    tbf = t_bucket.reshape(N_TOK).astype(jnp.int32)
    tyf = token_type.reshape(N_TOK).astype(jnp.int32)
    n_blk = N_TC // TL
    tb3 = tbf[:N_TC].reshape(n_blk, 1, TL)
    ty3 = tyf[:N_TC].reshape(n_blk, 1, TL)
    rb2 = router_b.reshape(1, E)

    lut_tb, lut_ty = _make_luts(t_embed_weight, type_embed_weight,
                                router_w, rb2)
    wh = router_w[:, :H]

    sc_out = _sc_router_call(h2, tbf, tyf, wh,
                             lut_tb.reshape(NUM_T_BUCKETS * E),
                             lut_ty.reshape(NUM_TOKEN_TYPES * E))
    tc_out = _tc_router(h2, tb3, ty3, t_embed_weight, type_embed_weight,
                        router_w, rb2)
    out = jnp.concatenate([tc_out, sc_out.reshape(N_SC, E)], axis=0)
    return out.reshape(B, L, E)


# speedup vs baseline: 1.5282x; 1.5282x over previous
import functools

import jax
import jax.numpy as jnp
from jax.experimental import pallas as pl

B, L, H, A, E = 4, 8192, 768, 64, 8
NUM_T_BUCKETS, NUM_TOKEN_TYPES = 8, 4
TL = 4096


def _router_block(h_ref, tb_ref, ty_ref, tw_ref, tyw_ref, rw_ref, rb_ref,
                  out_ref):
    h = h_ref[...]
    w_h = rw_ref[:, :H]
    logits = jax.lax.dot_general(
        h, w_h, (((1,), (1,)), ((), ())),
        preferred_element_type=jnp.float32)

    lut_t = jax.lax.dot_general(
        tw_ref[...], rw_ref[:, H:H + A], (((1,), (1,)), ((), ())),
        preferred_element_type=jnp.float32)
    lut_ty = jax.lax.dot_general(
        tyw_ref[...], rw_ref[:, H + A:], (((1,), (1,)), ((), ())),
        preferred_element_type=jnp.float32)

    tb = tb_ref[0, 0, :]
    ty = ty_ref[0, 0, :]
    oh_t = (tb[:, None] == jax.lax.broadcasted_iota(
        jnp.int32, (TL, NUM_T_BUCKETS), 1)).astype(jnp.float32)
    oh_ty = (ty[:, None] == jax.lax.broadcasted_iota(
        jnp.int32, (TL, NUM_TOKEN_TYPES), 1)).astype(jnp.float32)
    logits = logits + jax.lax.dot_general(
        oh_t, lut_t, (((1,), (0,)), ((), ())),
        preferred_element_type=jnp.float32)
    logits = logits + jax.lax.dot_general(
        oh_ty, lut_ty, (((1,), (0,)), ((), ())),
        preferred_element_type=jnp.float32)
    logits = logits + rb_ref[0, :][None, :]

    m = jnp.max(logits, axis=1, keepdims=True)
    e = jnp.exp(logits - m)
    out_ref[...] = e / jnp.sum(e, axis=1, keepdims=True)


@functools.partial(jax.jit, static_argnames=())
def kernel(hidden_states, t_bucket, token_type, t_embed_weight,
           type_embed_weight, router_w, router_b):
    n_tok = B * L
    n_blk = n_tok // TL
    h2 = hidden_states.reshape(n_tok, H)
    tb = t_bucket.reshape(n_blk, 1, TL).astype(jnp.int32)
    ty = token_type.reshape(n_blk, 1, TL).astype(jnp.int32)
    rb = router_b.reshape(1, E)

    out = pl.pallas_call(
        _router_block,
        grid=(n_blk,),
        in_specs=[
            pl.BlockSpec((TL, H), lambda i: (i, 0)),
            pl.BlockSpec((1, 1, TL), lambda i: (i, 0, 0)),
            pl.BlockSpec((1, 1, TL), lambda i: (i, 0, 0)),
            pl.BlockSpec((NUM_T_BUCKETS, A), lambda i: (0, 0)),
            pl.BlockSpec((NUM_TOKEN_TYPES, A), lambda i: (0, 0)),
            pl.BlockSpec((E, H + 2 * A), lambda i: (0, 0)),
            pl.BlockSpec((1, E), lambda i: (0, 0)),
        ],
        out_specs=pl.BlockSpec((TL, E), lambda i: (i, 0)),
        out_shape=jax.ShapeDtypeStruct((n_tok, E), jnp.float32),
    )(h2, tb, ty, t_embed_weight, type_embed_weight, router_w, rb)
    return out.reshape(B, L, E)
